# baseline (device time: 26716 ns/iter reference)
import jax
import jax.numpy as jnp
from jax import lax
from jax.experimental import pallas as pl
from jax.experimental.pallas import tpu as pltpu


def kernel(x, Wg, Wu, Wd):
    m, d = x.shape

    def body(x_ref, wg_ref, wu_ref, wd_ref, out_ref,
             send_buf, recv_buf, send_sems, recv_sems):
        my = lax.axis_index("i")
        p0 = my ^ 1
        p1 = 3 - my

        barrier_sem = pltpu.get_barrier_semaphore()
        pl.semaphore_signal(barrier_sem, inc=1, device_id=(p0,),
                            device_id_type=pl.DeviceIdType.MESH)
        pl.semaphore_signal(barrier_sem, inc=1, device_id=(p1,),
                            device_id_type=pl.DeviceIdType.MESH)
        pl.semaphore_wait(barrier_sem, 2)

        xb = x_ref[...].astype(jnp.bfloat16)
        gate = jnp.dot(xb, wg_ref[...].astype(jnp.bfloat16),
                       preferred_element_type=jnp.float32)
        up = jnp.dot(xb, wu_ref[...].astype(jnp.bfloat16),
                     preferred_element_type=jnp.float32)
        h = gate * (up * jax.nn.sigmoid(up))
        partial = jnp.dot(h.astype(jnp.bfloat16),
                          wd_ref[...].astype(jnp.bfloat16),
                          preferred_element_type=jnp.float32)

        send_buf[0] = partial.astype(jnp.bfloat16)
        rdma0 = pltpu.make_async_remote_copy(
            src_ref=send_buf.at[0], dst_ref=recv_buf.at[0],
            send_sem=send_sems.at[0], recv_sem=recv_sems.at[0],
            device_id=(p0,), device_id_type=pl.DeviceIdType.MESH,
        )
        rdma0.start()
        rdma0.wait()
        acc = partial + recv_buf[0].astype(jnp.float32)

        send_buf[1] = acc.astype(jnp.bfloat16)
        rdma1 = pltpu.make_async_remote_copy(
            src_ref=send_buf.at[1], dst_ref=recv_buf.at[1],
            send_sem=send_sems.at[1], recv_sem=recv_sems.at[1],
            device_id=(p1,), device_id_type=pl.DeviceIdType.MESH,
        )
        rdma1.start()
        rdma1.wait()
        out_ref[...] = acc + recv_buf[1].astype(jnp.float32)

    return pl.pallas_call(
        body,
        out_shape=jax.ShapeDtypeStruct((m, d), jnp.float32),
        in_specs=[pl.BlockSpec(memory_space=pltpu.VMEM)] * 4,
        out_specs=pl.BlockSpec(memory_space=pltpu.VMEM),
        scratch_shapes=[
            pltpu.VMEM((2, m, d), jnp.bfloat16),
            pltpu.VMEM((2, m, d), jnp.bfloat16),
            pltpu.SemaphoreType.DMA((2,)),
            pltpu.SemaphoreType.DMA((2,)),
        ],
        compiler_params=pltpu.CompilerParams(collective_id=0),
    )(x, Wg, Wu, Wd)


# device time: 22566 ns/iter; 1.1839x vs baseline; 1.1839x over previous
import jax
import jax.numpy as jnp
from jax import lax
from jax.experimental import pallas as pl
from jax.experimental.pallas import tpu as pltpu

N_CHUNKS = 4


def kernel(x, Wg, Wu, Wd):
    m, d = x.shape
    w = d // N_CHUNKS

    def body(x_ref, wg_ref, wu_ref, wd_ref, out_ref,
             send0, recv0, send1, recv1, s0_sems, r0_sems, s1_sems, r1_sems):
        my = lax.axis_index("i")
        p0 = my ^ 1
        p1 = 3 - my

        barrier_sem = pltpu.get_barrier_semaphore()
        pl.semaphore_signal(barrier_sem, inc=1, device_id=(p0,),
                            device_id_type=pl.DeviceIdType.MESH)
        pl.semaphore_signal(barrier_sem, inc=1, device_id=(p1,),
                            device_id_type=pl.DeviceIdType.MESH)
        pl.semaphore_wait(barrier_sem, 2)

        xb = x_ref[...].astype(jnp.bfloat16)
        gate = jnp.dot(xb, wg_ref[...].astype(jnp.bfloat16),
                       preferred_element_type=jnp.float32)
        up = jnp.dot(xb, wu_ref[...].astype(jnp.bfloat16),
                     preferred_element_type=jnp.float32)
        hb = (gate * (up * jax.nn.sigmoid(up))).astype(jnp.bfloat16)
        wd = wd_ref[...].astype(jnp.bfloat16)

        partials = []
        rdma0 = []
        for c in range(N_CHUNKS):
            partial = jnp.dot(hb, wd[:, c * w:(c + 1) * w],
                              preferred_element_type=jnp.float32)
            partials.append(partial)
            send0[c] = partial.astype(jnp.bfloat16)
            r = pltpu.make_async_remote_copy(
                src_ref=send0.at[c], dst_ref=recv0.at[c],
                send_sem=s0_sems.at[c], recv_sem=r0_sems.at[c],
                device_id=(p0,), device_id_type=pl.DeviceIdType.MESH,
            )
            r.start()
            rdma0.append(r)

        accs = []
        rdma1 = []
        for c in range(N_CHUNKS):
            rdma0[c].wait_recv()
            acc = partials[c] + recv0[c].astype(jnp.float32)
            accs.append(acc)
            send1[c] = acc.astype(jnp.bfloat16)
            r = pltpu.make_async_remote_copy(
                src_ref=send1.at[c], dst_ref=recv1.at[c],
                send_sem=s1_sems.at[c], recv_sem=r1_sems.at[c],
                device_id=(p1,), device_id_type=pl.DeviceIdType.MESH,
            )
            r.start()
            rdma1.append(r)

        for c in range(N_CHUNKS):
            rdma1[c].wait_recv()
            out_ref[:, c * w:(c + 1) * w] = accs[c] + recv1[c].astype(jnp.float32)

        for c in range(N_CHUNKS):
            rdma0[c].wait_send()
            rdma1[c].wait_send()

    return pl.pallas_call(
        body,
        out_shape=jax.ShapeDtypeStruct((m, d), jnp.float32),
        in_specs=[pl.BlockSpec(memory_space=pltpu.VMEM)] * 4,
        out_specs=pl.BlockSpec(memory_space=pltpu.VMEM),
        scratch_shapes=[
            pltpu.VMEM((N_CHUNKS, m, w), jnp.bfloat16),
            pltpu.VMEM((N_CHUNKS, m, w), jnp.bfloat16),
            pltpu.VMEM((N_CHUNKS, m, w), jnp.bfloat16),
            pltpu.VMEM((N_CHUNKS, m, w), jnp.bfloat16),
            pltpu.SemaphoreType.DMA((N_CHUNKS,)),
            pltpu.SemaphoreType.DMA((N_CHUNKS,)),
            pltpu.SemaphoreType.DMA((N_CHUNKS,)),
            pltpu.SemaphoreType.DMA((N_CHUNKS,)),
        ],
        compiler_params=pltpu.CompilerParams(collective_id=0),
    )(x, Wg, Wu, Wd)


# device time: 20223 ns/iter; 1.3211x vs baseline; 1.1159x over previous
import jax
import jax.numpy as jnp
from jax import lax
from jax.experimental import pallas as pl
from jax.experimental.pallas import tpu as pltpu

N_CHUNKS = 4


def kernel(x, Wg, Wu, Wd):
    m, d = x.shape
    w = d // N_CHUNKS

    def body(x_ref, wg_ref, wu_ref, wd_ref, out_ref,
             send0, recv0, send1, recv1, s0_sems, r0_sems, s1_sems, r1_sems):
        my = lax.axis_index("i")
        partners = [my ^ 1, 3 - my]

        barrier_sem = pltpu.get_barrier_semaphore()
        for p in partners:
            pl.semaphore_signal(barrier_sem, inc=1, device_id=(p,),
                                device_id_type=pl.DeviceIdType.MESH)
        pl.semaphore_wait(barrier_sem, 2)

        xb = x_ref[...].astype(jnp.bfloat16)
        gate = jnp.dot(xb, wg_ref[...].astype(jnp.bfloat16),
                       preferred_element_type=jnp.float32)
        up = jnp.dot(xb, wu_ref[...].astype(jnp.bfloat16),
                     preferred_element_type=jnp.float32)
        hb = (gate * (up * jax.nn.sigmoid(up))).astype(jnp.bfloat16)
        wd = wd_ref[...].astype(jnp.bfloat16)

        rdma0 = []
        for c in range(N_CHUNKS):
            partial = jnp.dot(hb, wd[:, c * w:(c + 1) * w],
                              preferred_element_type=jnp.float32)
            send0[c] = partial.astype(jnp.bfloat16)
            r = pltpu.make_async_remote_copy(
                src_ref=send0.at[c], dst_ref=recv0.at[c],
                send_sem=s0_sems.at[c], recv_sem=r0_sems.at[c],
                device_id=(partners[c % 2],),
                device_id_type=pl.DeviceIdType.MESH,
            )
            r.start()
            rdma0.append(r)

        rdma1 = []
        for c in range(N_CHUNKS):
            rdma0[c].wait_recv()
            send1[c] = (send0[c].astype(jnp.float32)
                        + recv0[c].astype(jnp.float32)).astype(jnp.bfloat16)
            r = pltpu.make_async_remote_copy(
                src_ref=send1.at[c], dst_ref=recv1.at[c],
                send_sem=s1_sems.at[c], recv_sem=r1_sems.at[c],
                device_id=(partners[1 - c % 2],),
                device_id_type=pl.DeviceIdType.MESH,
            )
            r.start()
            rdma1.append(r)

        for c in range(N_CHUNKS):
            rdma1[c].wait_recv()
            out_ref[:, c * w:(c + 1) * w] = (
                send1[c].astype(jnp.float32) + recv1[c].astype(jnp.float32))

        for c in range(N_CHUNKS):
            rdma0[c].wait_send()
            rdma1[c].wait_send()

    return pl.pallas_call(
        body,
        out_shape=jax.ShapeDtypeStruct((m, d), jnp.float32),
        in_specs=[pl.BlockSpec(memory_space=pltpu.VMEM)] * 4,
        out_specs=pl.BlockSpec(memory_space=pltpu.VMEM),
        scratch_shapes=[
            pltpu.VMEM((N_CHUNKS, m, w), jnp.bfloat16),
            pltpu.VMEM((N_CHUNKS, m, w), jnp.bfloat16),
            pltpu.VMEM((N_CHUNKS, m, w), jnp.bfloat16),
            pltpu.VMEM((N_CHUNKS, m, w), jnp.bfloat16),
            pltpu.SemaphoreType.DMA((N_CHUNKS,)),
            pltpu.SemaphoreType.DMA((N_CHUNKS,)),
            pltpu.SemaphoreType.DMA((N_CHUNKS,)),
            pltpu.SemaphoreType.DMA((N_CHUNKS,)),
        ],
        compiler_params=pltpu.CompilerParams(collective_id=0),
    )(x, Wg, Wu, Wd)


# device time: 20018 ns/iter; 1.3346x vs baseline; 1.0102x over previous
import jax
import jax.numpy as jnp
from jax import lax
from jax.experimental import pallas as pl
from jax.experimental.pallas import tpu as pltpu

N_CHUNKS = 4


def kernel(x, Wg, Wu, Wd):
    m, d = x.shape
    w = d // N_CHUNKS

    def body(x_ref, wg_ref, wu_ref, wd_ref, out_ref,
             send0, recv0, send1, recv1, s0_sems, r0_sems, s1_sems, r1_sems):
        my = lax.axis_index("i")
        partners = [my ^ 1, 3 - my]

        barrier_sem = pltpu.get_barrier_semaphore()
        for p in partners:
            pl.semaphore_signal(barrier_sem, inc=1, device_id=(p,),
                                device_id_type=pl.DeviceIdType.MESH)

        xb = x_ref[...].astype(jnp.bfloat16)
        gate = jnp.dot(xb, wg_ref[...].astype(jnp.bfloat16),
                       preferred_element_type=jnp.float32)
        up = jnp.dot(xb, wu_ref[...].astype(jnp.bfloat16),
                     preferred_element_type=jnp.float32)
        hb = (gate * (up * jax.nn.sigmoid(up))).astype(jnp.bfloat16)
        wd = wd_ref[...].astype(jnp.bfloat16)

        pl.semaphore_wait(barrier_sem, 2)

        rdma0 = []
        for c in range(N_CHUNKS):
            partial = jnp.dot(hb, wd[:, c * w:(c + 1) * w],
                              preferred_element_type=jnp.float32)
            send0[c] = partial.astype(jnp.bfloat16)
            r = pltpu.make_async_remote_copy(
                src_ref=send0.at[c], dst_ref=recv0.at[c],
                send_sem=s0_sems.at[c], recv_sem=r0_sems.at[c],
                device_id=(partners[c % 2],),
                device_id_type=pl.DeviceIdType.MESH,
            )
            r.start()
            rdma0.append(r)

        rdma1 = []
        for c in range(N_CHUNKS):
            rdma0[c].wait_recv()
            send1[c] = (send0[c].astype(jnp.float32)
                        + recv0[c].astype(jnp.float32)).astype(jnp.bfloat16)
            r = pltpu.make_async_remote_copy(
                src_ref=send1.at[c], dst_ref=recv1.at[c],
                send_sem=s1_sems.at[c], recv_sem=r1_sems.at[c],
                device_id=(partners[1 - c % 2],),
                device_id_type=pl.DeviceIdType.MESH,
            )
            r.start()
            rdma1.append(r)

        for c in range(N_CHUNKS):
            rdma1[c].wait_recv()
            out_ref[:, c * w:(c + 1) * w] = (
                send1[c].astype(jnp.float32) + recv1[c].astype(jnp.float32))

        for c in range(N_CHUNKS):
            rdma0[c].wait_send()
            rdma1[c].wait_send()

    return pl.pallas_call(
        body,
        out_shape=jax.ShapeDtypeStruct((m, d), jnp.float32),
        in_specs=[pl.BlockSpec(memory_space=pltpu.VMEM)] * 4,
        out_specs=pl.BlockSpec(memory_space=pltpu.VMEM),
        scratch_shapes=[
            pltpu.VMEM((N_CHUNKS, m, w), jnp.bfloat16),
            pltpu.VMEM((N_CHUNKS, m, w), jnp.bfloat16),
            pltpu.VMEM((N_CHUNKS, m, w), jnp.bfloat16),
            pltpu.VMEM((N_CHUNKS, m, w), jnp.bfloat16),
            pltpu.SemaphoreType.DMA((N_CHUNKS,)),
            pltpu.SemaphoreType.DMA((N_CHUNKS,)),
            pltpu.SemaphoreType.DMA((N_CHUNKS,)),
            pltpu.SemaphoreType.DMA((N_CHUNKS,)),
        ],
        compiler_params=pltpu.CompilerParams(collective_id=0),
    )(x, Wg, Wu, Wd)


# device time: 19815 ns/iter; 1.3483x vs baseline; 1.0102x over previous
import jax
import jax.numpy as jnp
from jax import lax
from jax.experimental import pallas as pl
from jax.experimental.pallas import tpu as pltpu

N_CHUNKS = 4


def kernel(x, Wg, Wu, Wd):
    m, d = x.shape
    w = d // N_CHUNKS

    def body(x_ref, wg_ref, wu_ref, wd_ref, out_ref,
             send0, recv0, send1, recv1, s0_sems, r0_sems, s1_sems, r1_sems):
        my = lax.axis_index("i")
        partners = [my ^ 1, 3 - my]

        barrier_sem = pltpu.get_barrier_semaphore()
        for p in partners:
            pl.semaphore_signal(barrier_sem, inc=1, device_id=(p,),
                                device_id_type=pl.DeviceIdType.MESH)

        xb = x_ref[...].astype(jnp.bfloat16)
        gate = jnp.dot(xb, wg_ref[...].astype(jnp.bfloat16),
                       preferred_element_type=jnp.float32)
        up = jnp.dot(xb, wu_ref[...].astype(jnp.bfloat16),
                     preferred_element_type=jnp.float32)
        hb = (gate * (up * jax.nn.sigmoid(up))).astype(jnp.bfloat16)
        wd = wd_ref[...].astype(jnp.bfloat16)

        pl.semaphore_wait(barrier_sem, 2)

        full_partial = jnp.dot(hb, wd, preferred_element_type=jnp.float32
                               ).astype(jnp.bfloat16)
        rdma0 = []
        for c in range(N_CHUNKS):
            send0[c] = full_partial[:, c * w:(c + 1) * w]
            r = pltpu.make_async_remote_copy(
                src_ref=send0.at[c], dst_ref=recv0.at[c],
                send_sem=s0_sems.at[c], recv_sem=r0_sems.at[c],
                device_id=(partners[c % 2],),
                device_id_type=pl.DeviceIdType.MESH,
            )
            r.start()
            rdma0.append(r)

        rdma1 = []
        for c in range(N_CHUNKS):
            rdma0[c].wait_recv()
            send1[c] = send0[c] + recv0[c]
            r = pltpu.make_async_remote_copy(
                src_ref=send1.at[c], dst_ref=recv1.at[c],
                send_sem=s1_sems.at[c], recv_sem=r1_sems.at[c],
                device_id=(partners[1 - c % 2],),
                device_id_type=pl.DeviceIdType.MESH,
            )
            r.start()
            rdma1.append(r)

        for c in range(N_CHUNKS):
            rdma1[c].wait_recv()
            out_ref[:, c * w:(c + 1) * w] = (
                send1[c].astype(jnp.float32) + recv1[c].astype(jnp.float32))

        for c in range(N_CHUNKS):
            rdma0[c].wait_send()
            rdma1[c].wait_send()

    return pl.pallas_call(
        body,
        out_shape=jax.ShapeDtypeStruct((m, d), jnp.float32),
        in_specs=[pl.BlockSpec(memory_space=pltpu.VMEM)] * 4,
        out_specs=pl.BlockSpec(memory_space=pltpu.VMEM),
        scratch_shapes=[
            pltpu.VMEM((N_CHUNKS, m, w), jnp.bfloat16),
            pltpu.VMEM((N_CHUNKS, m, w), jnp.bfloat16),
            pltpu.VMEM((N_CHUNKS, m, w), jnp.bfloat16),
            pltpu.VMEM((N_CHUNKS, m, w), jnp.bfloat16),
            pltpu.SemaphoreType.DMA((N_CHUNKS,)),
            pltpu.SemaphoreType.DMA((N_CHUNKS,)),
            pltpu.SemaphoreType.DMA((N_CHUNKS,)),
            pltpu.SemaphoreType.DMA((N_CHUNKS,)),
        ],
        compiler_params=pltpu.CompilerParams(collective_id=0),
    )(x, Wg, Wu, Wd)
